# Initial kernel scaffold; baseline (speedup 1.0000x reference)
import jax
import jax.numpy as jnp
from jax.experimental import pallas as pl


def kernel(x, edge_index, W1, b1, W2, b2, W3, b3):
    raise NotImplementedError("write your pallas kernel here")


# speedup vs baseline: 13.0163x; 13.0163x over previous
import functools

import jax
import jax.numpy as jnp
from jax import lax
from jax.experimental import pallas as pl
from jax.experimental.pallas import tpu as pltpu
from jax.experimental.pallas import tpu_sc as plsc

N_NODES = 10000
N_EDGES = 320000
D_FEAT = 128
HIDDEN = 128
N_CLASSES = 10
CPAD = 16

NC, NS, L = 2, 16, 16
NW = NC * NS
NPAD = 10240
RPS = NPAD // NS
B = 128
NB = -(-N_EDGES // (NW * B))
EPAD = NW * NB * B

F32 = jnp.float32


def _sc_mesh():
    return plsc.VectorSubcoreMesh(core_axis_name="c", subcore_axis_name="s")


@functools.partial(
    pl.kernel,
    out_type=jax.ShapeDtypeStruct((NW, NPAD), F32),
    mesh=_sc_mesh(),
    compiler_params=pltpu.CompilerParams(needs_layout_passes=False, use_tc_tiling_on_sc=False),
    scratch_types=[
        pltpu.VMEM((NB, B), jnp.int32),
        pltpu.VMEM((NPAD,), F32),
    ],
)
def _deg_kernel(dst_hbm, out_hbm, dst_v, cnt_v):
    c = lax.axis_index("c")
    s = lax.axis_index("s")
    wid = s * NC + c
    pltpu.sync_copy(dst_hbm.at[wid], dst_v)

    zero = jnp.zeros((L,), F32)

    def zbody(i, _):
        cnt_v[pl.ds(i * L, L)] = zero
        return 0

    lax.fori_loop(0, NPAD // L, zbody, 0)

    ones = jnp.ones((L,), F32)

    def body(j, _):
        for t in range(B // L):
            idx = dst_v[j, pl.ds(t * L, L)]
            plsc.addupdate_scatter(cnt_v, [idx], ones)
        return 0

    lax.fori_loop(0, NB, body, 0)
    pltpu.sync_copy(cnt_v, out_hbm.at[wid])


@functools.lru_cache(maxsize=None)
def _make_agg(F):
    @functools.partial(
        pl.kernel,
        out_type=jax.ShapeDtypeStruct((NC, NPAD, F), F32),
        mesh=_sc_mesh(),
        compiler_params=pltpu.CompilerParams(needs_layout_passes=False, use_tc_tiling_on_sc=False),
        scratch_types=[
            pltpu.VMEM((NB, B), jnp.int32),
            pltpu.VMEM((NB, B), jnp.int32),
            pltpu.VMEM((B, F), F32),
            pltpu.VMEM_SHARED((NPAD, F), F32),
        ],
    )
    def agg_kernel(y_hbm, src_hbm, dst_hbm, out_hbm, src_v, dst_v, rows_v, acc_sh):
        c = lax.axis_index("c")
        s = lax.axis_index("s")
        wid = s * NC + c
        pltpu.sync_copy(src_hbm.at[wid], src_v)
        pltpu.sync_copy(dst_hbm.at[wid], dst_v)

        zero = jnp.zeros((L,), F32)

        def zbody(r, _):
            for t in range(F // L):
                rows_v[r, pl.ds(t * L, L)] = zero
            return 0

        lax.fori_loop(0, B, zbody, 0)
        for chunk in range(RPS // B):
            pltpu.sync_copy(rows_v, acc_sh.at[pl.ds(s * RPS + chunk * B, B)])
        plsc.subcore_barrier()

        def body(j, _):
            pltpu.sync_copy(y_hbm.at[src_v.at[j]], rows_v)
            pltpu.sync_copy(rows_v, acc_sh.at[dst_v.at[j]], add=True)
            return 0

        lax.fori_loop(0, NB, body, 0)
        plsc.subcore_barrier()
        pltpu.sync_copy(acc_sh.at[pl.ds(s * RPS, RPS)],
                        out_hbm.at[c, pl.ds(s * RPS, RPS)])

    return agg_kernel


_R = 1024
_GRID = NPAD // _R


def _elu(v):
    return jnp.where(v > 0, v, jnp.exp(jnp.minimum(v, 0.0)) - 1.0)


def _b1_body(x_ref, w_ref, degp_ref, y_ref, dinv_ref):
    deg = 1.0 + jnp.sum(degp_ref[...], axis=0)
    dinv = lax.rsqrt(deg)
    xw = jnp.dot(x_ref[...], w_ref[...], preferred_element_type=F32)
    y_ref[...] = xw * dinv[:, None]
    dinv_ref[...] = dinv[:, None]


def _stage1(xp, W1, degp):
    return pl.pallas_call(
        _b1_body,
        grid=(_GRID,),
        in_specs=[
            pl.BlockSpec((_R, D_FEAT), lambda i: (i, 0)),
            pl.BlockSpec((D_FEAT, HIDDEN), lambda i: (0, 0)),
            pl.BlockSpec((NW, _R), lambda i: (0, i)),
        ],
        out_specs=[
            pl.BlockSpec((_R, HIDDEN), lambda i: (i, 0)),
            pl.BlockSpec((_R, 1), lambda i: (i, 0)),
        ],
        out_shape=[
            jax.ShapeDtypeStruct((NPAD, HIDDEN), F32),
            jax.ShapeDtypeStruct((NPAD, 1), F32),
        ],
    )(xp, W1, degp)


def _b2_body(p0_ref, p1_ref, y1_ref, dinv_ref, b1_ref, w2_ref, x1_ref, y2_ref):
    dinv = dinv_ref[...]
    pre = dinv * (p0_ref[...] + p1_ref[...] + y1_ref[...]) + b1_ref[...]
    x1 = _elu(pre)
    x1_ref[...] = x1
    y2_ref[...] = dinv * jnp.dot(x1, w2_ref[...], preferred_element_type=F32)


def _stage2(p0, p1, y1, dinv, b1, W2):
    return pl.pallas_call(
        _b2_body,
        grid=(_GRID,),
        in_specs=[
            pl.BlockSpec((_R, HIDDEN), lambda i: (i, 0)),
            pl.BlockSpec((_R, HIDDEN), lambda i: (i, 0)),
            pl.BlockSpec((_R, HIDDEN), lambda i: (i, 0)),
            pl.BlockSpec((_R, 1), lambda i: (i, 0)),
            pl.BlockSpec((1, HIDDEN), lambda i: (0, 0)),
            pl.BlockSpec((HIDDEN, HIDDEN), lambda i: (0, 0)),
        ],
        out_specs=[
            pl.BlockSpec((_R, HIDDEN), lambda i: (i, 0)),
            pl.BlockSpec((_R, HIDDEN), lambda i: (i, 0)),
        ],
        out_shape=[
            jax.ShapeDtypeStruct((NPAD, HIDDEN), F32),
            jax.ShapeDtypeStruct((NPAD, HIDDEN), F32),
        ],
    )(p0, p1, y1, dinv, b1, W2)


def _b3_body(p0_ref, p1_ref, y2_ref, x1_ref, dinv_ref, b2_ref, w3_ref, y3_ref):
    dinv = dinv_ref[...]
    pre = dinv * (p0_ref[...] + p1_ref[...] + y2_ref[...]) + b2_ref[...] + x1_ref[...]
    x2 = _elu(pre)
    y3_ref[...] = dinv * jnp.dot(x2, w3_ref[...], preferred_element_type=F32)


def _stage3(p0, p1, y2, x1, dinv, b2, W3p):
    return pl.pallas_call(
        _b3_body,
        grid=(_GRID,),
        in_specs=[
            pl.BlockSpec((_R, HIDDEN), lambda i: (i, 0)),
            pl.BlockSpec((_R, HIDDEN), lambda i: (i, 0)),
            pl.BlockSpec((_R, HIDDEN), lambda i: (i, 0)),
            pl.BlockSpec((_R, HIDDEN), lambda i: (i, 0)),
            pl.BlockSpec((_R, 1), lambda i: (i, 0)),
            pl.BlockSpec((1, HIDDEN), lambda i: (0, 0)),
            pl.BlockSpec((HIDDEN, CPAD), lambda i: (0, 0)),
        ],
        out_specs=pl.BlockSpec((_R, CPAD), lambda i: (i, 0)),
        out_shape=jax.ShapeDtypeStruct((NPAD, CPAD), F32),
    )(p0, p1, y2, x1, dinv, b2, W3p)


def _b4_body(p0_ref, p1_ref, y3_ref, dinv_ref, b3_ref, out_ref):
    out_ref[...] = (dinv_ref[...] * (p0_ref[...] + p1_ref[...] + y3_ref[...])
                    + b3_ref[...])


def _stage4(p0, p1, y3, dinv, b3p):
    return pl.pallas_call(
        _b4_body,
        grid=(_GRID,),
        in_specs=[
            pl.BlockSpec((_R, CPAD), lambda i: (i, 0)),
            pl.BlockSpec((_R, CPAD), lambda i: (i, 0)),
            pl.BlockSpec((_R, CPAD), lambda i: (i, 0)),
            pl.BlockSpec((_R, 1), lambda i: (i, 0)),
            pl.BlockSpec((1, CPAD), lambda i: (0, 0)),
        ],
        out_specs=pl.BlockSpec((_R, CPAD), lambda i: (i, 0)),
        out_shape=jax.ShapeDtypeStruct((NPAD, CPAD), F32),
    )(p0, p1, y3, dinv, b3p)


def kernel(x, edge_index, W1, b1, W2, b2, W3, b3):
    ei = edge_index.astype(jnp.int32)
    pad = jnp.full((EPAD - N_EDGES,), N_NODES, jnp.int32)
    srcp = jnp.concatenate([ei[0], pad]).reshape(NW, NB, B)
    dstp = jnp.concatenate([ei[1], pad]).reshape(NW, NB, B)

    xp = jnp.zeros((NPAD, D_FEAT), F32).at[:N_NODES].set(x)
    W3p = jnp.zeros((HIDDEN, CPAD), F32).at[:, :N_CLASSES].set(W3)
    b3p = jnp.zeros((1, CPAD), F32).at[0, :N_CLASSES].set(b3)

    degp = _deg_kernel(dstp)
    y1, dinv = _stage1(xp, W1, degp)

    agg128 = _make_agg(HIDDEN)
    p1 = agg128(y1, srcp, dstp)
    x1, y2 = _stage2(p1[0], p1[1], y1, dinv, b1.reshape(1, HIDDEN), W2)

    p2 = agg128(y2, srcp, dstp)
    y3 = _stage3(p2[0], p2[1], y2, x1, dinv, b2.reshape(1, HIDDEN), W3p)

    p3 = _make_agg(CPAD)(y3, srcp, dstp)
    out = _stage4(p3[0], p3[1], y3, dinv, b3p)
    return out[:N_NODES, :N_CLASSES]
